# baseline (speedup 1.0000x reference)
import math

import jax
import jax.numpy as jnp
from jax import lax
from jax.experimental import pallas as pl
from jax.experimental.pallas import tpu as pltpu
from jax.experimental.pallas import tpu_sc as plsc

N = 10000
E = 160000
HID = 256
HEADS = 8
D = HID // HEADS
QW = 64
HPQ = 2

NC = 2
NS = 16
LANES = 16

CHUNK = 80
EPW = E // NS
NCHUNK = EPW // CHUNK
NDRAIN = N // CHUNK

_ROW_BLK = 1000


def _pre_body(x_ref, linw_ref, linb_ref, lng_ref, lnb_ref, qw_ref, qb_ref,
              kw_ref, kb_ref, vw_ref, vb_ref,
              h_ref, q_ref, kv_ref):
    xb = x_ref[...]
    h = jnp.dot(xb, linw_ref[...], preferred_element_type=jnp.float32)
    h = h + linb_ref[...]
    mu = jnp.mean(h, axis=-1, keepdims=True)
    var = jnp.mean((h - mu) ** 2, axis=-1, keepdims=True)
    h = (h - mu) / jnp.sqrt(var + 1e-5) * lng_ref[...] + lnb_ref[...]
    h = jnp.maximum(h, 0.0)
    h_ref[...] = h
    q = jnp.dot(h, qw_ref[...], preferred_element_type=jnp.float32) + qb_ref[...]
    k = jnp.dot(h, kw_ref[...], preferred_element_type=jnp.float32) + kb_ref[...]
    v = jnp.dot(h, vw_ref[...], preferred_element_type=jnp.float32) + vb_ref[...]
    for qt in range(4):
        q_ref[qt] = q[:, qt * QW:(qt + 1) * QW]
        kv_ref[qt] = jnp.concatenate(
            [k[:, qt * QW:(qt + 1) * QW], v[:, qt * QW:(qt + 1) * QW]],
            axis=1)


def _pre(x, linw, linb, lng, lnb, qw, qb, kw, kb, vw, vb):
    grid = (N // _ROW_BLK,)
    full = lambda shape: pl.BlockSpec(shape, lambda i: tuple(0 for _ in shape))
    row = lambda w: pl.BlockSpec((_ROW_BLK, w), lambda i: (i, 0))
    quarter = pl.BlockSpec((4, _ROW_BLK, QW), lambda i: (0, i, 0))
    return pl.pallas_call(
        _pre_body,
        grid=grid,
        in_specs=[row(256), full((256, 256)), full((256,)), full((256,)),
                  full((256,)), full((256, 256)), full((256,)),
                  full((256, 256)), full((256,)), full((256, 256)),
                  full((256,))],
        out_specs=[row(256), quarter,
                   pl.BlockSpec((4, _ROW_BLK, 2 * QW), lambda i: (0, i, 0))],
        out_shape=[
            jax.ShapeDtypeStruct((N, HID), jnp.float32),
            jax.ShapeDtypeStruct((4, N, QW), jnp.float32),
            jax.ShapeDtypeStruct((4, N, 2 * QW), jnp.float32),
        ],
    )(x, linw, linb, lng, lnb, qw, qb, kw, kb, vw, vb)


def _edge_body(src, dst, qs, kvs, zn, zd, num, den,
               src_all, dst_all, sidx, didx, scidx, qrows, kvrows,
               msg, exb, num_s, den_s, sem_g0, sem_g1, sem_s0, sem_s1):
    c = lax.axis_index("c")
    s = lax.axis_index("s")
    lane = lax.iota(jnp.int32, LANES)
    sem_g = [sem_g0, sem_g1]
    sem_s = [sem_s0, sem_s1]

    def copy_idx(src_ref, dst_ref, j):
        for g in range(CHUNK // LANES):
            dst_ref[pl.ds(g * LANES, LANES)] = (
                src_ref[pl.ds(j * CHUNK + g * LANES, LANES)])

    def issue_gathers(qq, j, b):
        copy_idx(src_all, sidx[b], j)
        copy_idx(dst_all, didx[b], j)
        pltpu.async_copy(qs.at[qq].at[didx[b]], qrows[b], sem_g[b])
        pltpu.async_copy(kvs.at[qq].at[sidx[b]], kvrows[b], sem_g[b])

    def wait_gathers(qq, b):
        pltpu.make_async_copy(qs.at[qq].at[didx[b]], qrows[b], sem_g[b]).wait()
        pltpu.make_async_copy(kvs.at[qq].at[sidx[b]], kvrows[b],
                              sem_g[b]).wait()

    def wait_scatters(b):
        pltpu.make_async_copy(msg[b], num_s.at[scidx[b]], sem_s[b]).wait()
        pltpu.make_async_copy(exb[b], den_s.at[scidx[b]], sem_s[b]).wait()

    def process(qq, j, b, nb, prefetch, scatter_guard):
        wait_gathers(qq, b)
        if prefetch:
            issue_gathers(qq, j + 1, nb)
        if scatter_guard is True:
            wait_scatters(b)
        elif scatter_guard is not False:
            @pl.when(scatter_guard)
            def _():
                wait_scatters(b)

        def edges(e8, _):
            for i in range(8):
                e = e8 * 8 + i
                den_acc = jnp.zeros((LANES,), jnp.float32)
                for h in range(HPQ):
                    q0 = qrows[b][e, pl.ds(h * D, LANES)]
                    q1 = qrows[b][e, pl.ds(h * D + LANES, LANES)]
                    k0 = kvrows[b][e, pl.ds(h * D, LANES)]
                    k1 = kvrows[b][e, pl.ds(h * D + LANES, LANES)]
                    pr = q0 * k0 + q1 * k1
                    a = jnp.sum(pr)
                    bc = jnp.zeros((LANES,), jnp.float32) + a
                    bc = jnp.minimum(jnp.maximum(bc, -60.0), 60.0)
                    eb = jnp.exp(bc)
                    msg[b][e, pl.ds(h * D, LANES)] = (
                        kvrows[b][e, pl.ds(QW + h * D, LANES)] * eb)
                    msg[b][e, pl.ds(h * D + LANES, LANES)] = (
                        kvrows[b][e, pl.ds(QW + h * D + LANES, LANES)] * eb)
                    den_acc = den_acc + jnp.where(lane == h, eb, 0.0)
                exb[b][e, :] = den_acc
            return 0

        lax.fori_loop(0, CHUNK // 8, edges, 0)
        copy_idx(didx[b], scidx[b], 0)
        pltpu.async_copy(msg[b], num_s.at[scidx[b]], sem_s[b], add=True)
        pltpu.async_copy(exb[b], den_s.at[scidx[b]], sem_s[b], add=True)

    for p in range(2):
        qq = c * 2 + p

        for t in range((NDRAIN + NS - 1) // NS):
            cid = s + NS * t

            @pl.when(cid < NDRAIN)
            def _():
                rows = pl.ds(cid * CHUNK, CHUNK)
                pltpu.sync_copy(zn, num_s.at[rows])
                pltpu.sync_copy(zd, den_s.at[rows])

        plsc.subcore_barrier()

        pltpu.sync_copy(src.at[pl.ds(s * EPW, EPW)], src_all)
        pltpu.sync_copy(dst.at[pl.ds(s * EPW, EPW)], dst_all)

        issue_gathers(qq, 0, 0)

        def pair(pp, _):
            j0 = pp * 2
            guard = pp >= 1
            process(qq, j0, 0, 1, True, guard)
            process(qq, j0 + 1, 1, 0, True, guard)
            return 0

        lax.fori_loop(0, NCHUNK // 2, pair, 0)
        process(qq, NCHUNK - 1, 0, 1, False, True)
        wait_scatters(0)
        wait_scatters(1)

        plsc.subcore_barrier()

        for t in range((NDRAIN + NS - 1) // NS):
            cid = s + NS * t

            @pl.when(cid < NDRAIN)
            def _():
                rows = pl.ds(cid * CHUNK, CHUNK)
                pltpu.sync_copy(num_s.at[rows], msg[0])
                pltpu.sync_copy(msg[0], num.at[qq].at[rows])
                pltpu.sync_copy(den_s.at[rows], exb[0])
                pltpu.sync_copy(exb[0], den.at[qq].at[rows])

        plsc.subcore_barrier()


def _edge_pass(src, dst, qs, kvs):
    zn = jnp.zeros((CHUNK, QW), jnp.float32)
    zd = jnp.zeros((CHUNK, LANES), jnp.float32)
    mesh = plsc.VectorSubcoreMesh(core_axis_name="c", subcore_axis_name="s")
    f = pl.kernel(
        _edge_body,
        out_type=[
            jax.ShapeDtypeStruct((4, N, QW), jnp.float32),
            jax.ShapeDtypeStruct((4, N, LANES), jnp.float32),
        ],
        mesh=mesh,
        compiler_params=pltpu.CompilerParams(needs_layout_passes=False,
                                             use_tc_tiling_on_sc=False),
        scratch_types=[
            pltpu.VMEM((EPW,), jnp.int32),
            pltpu.VMEM((EPW,), jnp.int32),
            [pltpu.VMEM((CHUNK,), jnp.int32) for _ in range(2)],
            [pltpu.VMEM((CHUNK,), jnp.int32) for _ in range(2)],
            [pltpu.VMEM((CHUNK,), jnp.int32) for _ in range(2)],
            [pltpu.VMEM((CHUNK, QW), jnp.float32) for _ in range(2)],
            [pltpu.VMEM((CHUNK, 2 * QW), jnp.float32) for _ in range(2)],
            [pltpu.VMEM((CHUNK, QW), jnp.float32) for _ in range(2)],
            [pltpu.VMEM((CHUNK, LANES), jnp.float32) for _ in range(2)],
            pltpu.VMEM_SHARED((N, QW), jnp.float32),
            pltpu.VMEM_SHARED((N, LANES), jnp.float32),
            pltpu.SemaphoreType.DMA,
            pltpu.SemaphoreType.DMA,
            pltpu.SemaphoreType.DMA,
            pltpu.SemaphoreType.DMA,
        ],
    )
    return f(src, dst, qs, kvs, zn, zd)


def _post_body(num_ref, den_ref, h_ref, w_ref, b_ref, hc_ref,
               g_ref, bb_ref, out_ref):
    parts = []
    for qt in range(4):
        d2 = den_ref[qt][:, :HPQ]
        div = jnp.broadcast_to(d2[:, :, None], (_ROW_BLK, HPQ, D))
        div = div.reshape(_ROW_BLK, QW)
        parts.append(num_ref[qt] / (div + 1e-16))
    agg = jnp.concatenate(parts, axis=1)
    ge = agg * 0.5 * (1.0 + lax.erf(agg / math.sqrt(2.0)))
    out = jnp.dot(ge, w_ref[...], preferred_element_type=jnp.float32)
    out = out + b_ref[...] + hc_ref[0, 0] * h_ref[...]
    mu = jnp.mean(out, axis=-1, keepdims=True)
    var = jnp.mean((out - mu) ** 2, axis=-1, keepdims=True)
    out_ref[...] = (out - mu) / jnp.sqrt(var + 1e-5) * g_ref[...] + bb_ref[...]


def _post(num, den, h, w2, b2, hc, ng, nb):
    grid = (N // _ROW_BLK,)
    full = lambda shape: pl.BlockSpec(shape, lambda i: tuple(0 for _ in shape))
    return pl.pallas_call(
        _post_body,
        grid=grid,
        in_specs=[
            pl.BlockSpec((4, _ROW_BLK, QW), lambda i: (0, i, 0)),
            pl.BlockSpec((4, _ROW_BLK, LANES), lambda i: (0, i, 0)),
            pl.BlockSpec((_ROW_BLK, HID), lambda i: (i, 0)),
            full((256, 256)), full((256,)), full((1, 1)),
            full((256,)), full((256,)),
        ],
        out_specs=pl.BlockSpec((_ROW_BLK, HID), lambda i: (i, 0)),
        out_shape=jax.ShapeDtypeStruct((N, HID), jnp.float32),
    )(num, den, h, w2, b2, hc, ng, nb)


def kernel(x, edge_index, lin_W, lin_b, ln_in_g, ln_in_b, k_W, k_b, q_W, q_b,
           v_W, v_b, a_rel, m_rel, p_rel, skip, a_lin_W, a_lin_b,
           norm_g, norm_b):
    scale = p_rel / math.sqrt(D)
    qw = (q_W.T.reshape(HID, HEADS, D) * scale[None, :, None]).reshape(HID, HID)
    qb = (q_b.reshape(HEADS, D) * scale[:, None]).reshape(HID)
    kw = jnp.einsum("fhd,hde->fhe", k_W.T.reshape(HID, HEADS, D),
                    a_rel).reshape(HID, HID)
    kb = jnp.einsum("hd,hde->he", k_b.reshape(HEADS, D), a_rel).reshape(HID)
    vw = jnp.einsum("fhd,hde->fhe", v_W.T.reshape(HID, HEADS, D),
                    m_rel).reshape(HID, HID)
    vb = jnp.einsum("hd,hde->he", v_b.reshape(HEADS, D), m_rel).reshape(HID)

    h, qs, kvs = _pre(x, lin_W.T, lin_b, ln_in_g, ln_in_b,
                      qw, qb, kw, kb, vw, vb)

    src = edge_index[0]
    dst = edge_index[1]
    num, den = _edge_pass(src, dst, qs, kvs)

    a_skip = jax.nn.sigmoid(skip[0])
    w2 = a_lin_W.T * a_skip
    b2 = a_lin_b * a_skip
    hc = jnp.full((1, 1), 2.0 - a_skip, jnp.float32)

    return _post(num, den, h, w2, b2, hc, norm_g, norm_b)


# speedup vs baseline: 1.4797x; 1.4797x over previous
import math

import jax
import jax.numpy as jnp
from jax import lax
from jax.experimental import pallas as pl
from jax.experimental.pallas import tpu as pltpu
from jax.experimental.pallas import tpu_sc as plsc

N = 10000
E = 160000
HID = 256
HEADS = 8
D = HID // HEADS
QW = 64
HPQ = 2

NC = 2
NS = 16
LANES = 16

CHUNK = 80
EPW = E // NS
NCHUNK = EPW // CHUNK
NDRAIN = N // CHUNK

_ROW_BLK = 1000


def _pre_body(x_ref, linw_ref, linb_ref, lng_ref, lnb_ref, qw_ref, qb_ref,
              kw_ref, kb_ref, vw_ref, vb_ref,
              h_ref, q_ref, k_ref, v_ref):
    xb = x_ref[...]
    h = jnp.dot(xb, linw_ref[...], preferred_element_type=jnp.float32)
    h = h + linb_ref[...]
    mu = jnp.mean(h, axis=-1, keepdims=True)
    var = jnp.mean((h - mu) ** 2, axis=-1, keepdims=True)
    h = (h - mu) / jnp.sqrt(var + 1e-5) * lng_ref[...] + lnb_ref[...]
    h = jnp.maximum(h, 0.0)
    h_ref[...] = h
    q = jnp.dot(h, qw_ref[...], preferred_element_type=jnp.float32) + qb_ref[...]
    k = jnp.dot(h, kw_ref[...], preferred_element_type=jnp.float32) + kb_ref[...]
    v = jnp.dot(h, vw_ref[...], preferred_element_type=jnp.float32) + vb_ref[...]
    for qt in range(4):
        q_ref[qt] = q[:, qt * QW:(qt + 1) * QW]
        k_ref[qt] = k[:, qt * QW:(qt + 1) * QW]
        v_ref[qt] = v[:, qt * QW:(qt + 1) * QW]


def _pre(x, linw, linb, lng, lnb, qw, qb, kw, kb, vw, vb):
    grid = (N // _ROW_BLK,)
    full = lambda shape: pl.BlockSpec(shape, lambda i: tuple(0 for _ in shape))
    row = lambda w: pl.BlockSpec((_ROW_BLK, w), lambda i: (i, 0))
    quarter = pl.BlockSpec((4, _ROW_BLK, QW), lambda i: (0, i, 0))
    return pl.pallas_call(
        _pre_body,
        grid=grid,
        in_specs=[row(256), full((256, 256)), full((256,)), full((256,)),
                  full((256,)), full((256, 256)), full((256,)),
                  full((256, 256)), full((256,)), full((256, 256)),
                  full((256,))],
        out_specs=[row(256), quarter, quarter, quarter],
        out_shape=[
            jax.ShapeDtypeStruct((N, HID), jnp.float32),
            jax.ShapeDtypeStruct((4, N, QW), jnp.float32),
            jax.ShapeDtypeStruct((4, N, QW), jnp.float32),
            jax.ShapeDtypeStruct((4, N, QW), jnp.float32),
        ],
    )(x, linw, linb, lng, lnb, qw, qb, kw, kb, vw, vb)


def _edge_body(src, dst, qs, ks, vs, zn, zd, num, den,
               src_all, dst_all, sidx, didx, qrows, krows, vrows,
               msg, exb, num_s, den_s, sem_g0, sem_g1, sem_g2):
    c = lax.axis_index("c")
    s = lax.axis_index("s")
    lane = lax.iota(jnp.int32, LANES)
    sem_g = [sem_g0, sem_g1, sem_g2]

    def copy_idx(src_ref, dst_ref, j):
        for g in range(CHUNK // LANES):
            dst_ref[pl.ds(g * LANES, LANES)] = (
                src_ref[pl.ds(j * CHUNK + g * LANES, LANES)])

    def issue_gathers(qq, j, b):
        copy_idx(src_all, sidx[b], j)
        copy_idx(dst_all, didx[b], j)
        pltpu.async_copy(qs.at[qq].at[didx[b]], qrows[b], sem_g[b])
        pltpu.async_copy(ks.at[qq].at[sidx[b]], krows[b], sem_g[b])
        pltpu.async_copy(vs.at[qq].at[sidx[b]], vrows[b], sem_g[b])

    def wait_gathers(qq, b):
        pltpu.make_async_copy(qs.at[qq].at[didx[b]], qrows[b], sem_g[b]).wait()
        pltpu.make_async_copy(ks.at[qq].at[sidx[b]], krows[b], sem_g[b]).wait()
        pltpu.make_async_copy(vs.at[qq].at[sidx[b]], vrows[b], sem_g[b]).wait()

    def process(qq, j, b, nb, prefetch):
        wait_gathers(qq, b)
        if prefetch:
            issue_gathers(qq, j + 2, nb)

        def edges(e8, _):
            for i in range(8):
                e = e8 * 8 + i
                den_acc = jnp.zeros((LANES,), jnp.float32)
                for h in range(HPQ):
                    q0 = qrows[b][e, pl.ds(h * D, LANES)]
                    q1 = qrows[b][e, pl.ds(h * D + LANES, LANES)]
                    k0 = krows[b][e, pl.ds(h * D, LANES)]
                    k1 = krows[b][e, pl.ds(h * D + LANES, LANES)]
                    pr = q0 * k0 + q1 * k1
                    a = jnp.sum(pr)
                    bc = jnp.zeros((LANES,), jnp.float32) + a
                    bc = jnp.minimum(jnp.maximum(bc, -60.0), 60.0)
                    eb = jnp.exp(bc)
                    msg[e, pl.ds(h * D, LANES)] = (
                        vrows[b][e, pl.ds(h * D, LANES)] * eb)
                    msg[e, pl.ds(h * D + LANES, LANES)] = (
                        vrows[b][e, pl.ds(h * D + LANES, LANES)] * eb)
                    den_acc = den_acc + jnp.where(lane == h, eb, 0.0)
                exb[e, :] = den_acc
            return 0

        lax.fori_loop(0, CHUNK // 8, edges, 0)
        pltpu.sync_copy(msg, num_s.at[didx[b]], add=True)
        pltpu.sync_copy(exb, den_s.at[didx[b]], add=True)

    for p in range(2):
        qq = c * 2 + p

        for t in range((NDRAIN + NS - 1) // NS):
            cid = s + NS * t

            @pl.when(cid < NDRAIN)
            def _():
                rows = pl.ds(cid * CHUNK, CHUNK)
                pltpu.sync_copy(zn, num_s.at[rows])
                pltpu.sync_copy(zd, den_s.at[rows])

        plsc.subcore_barrier()

        pltpu.sync_copy(src.at[pl.ds(s * EPW, EPW)], src_all)
        pltpu.sync_copy(dst.at[pl.ds(s * EPW, EPW)], dst_all)

        issue_gathers(qq, 0, 0)
        issue_gathers(qq, 1, 1)

        def triple(tr, _):
            j0 = tr * 3
            process(qq, j0, 0, 2, True)
            process(qq, j0 + 1, 1, 0, True)
            process(qq, j0 + 2, 2, 1, True)
            return 0

        lax.fori_loop(0, NCHUNK // 3, triple, 0)
        process(qq, NCHUNK - 2, 0, 2, False)
        process(qq, NCHUNK - 1, 1, 0, False)

        plsc.subcore_barrier()

        for t in range((NDRAIN + NS - 1) // NS):
            cid = s + NS * t

            @pl.when(cid < NDRAIN)
            def _():
                rows = pl.ds(cid * CHUNK, CHUNK)
                pltpu.sync_copy(num_s.at[rows], msg)
                pltpu.sync_copy(msg, num.at[qq].at[rows])
                pltpu.sync_copy(den_s.at[rows], exb)
                pltpu.sync_copy(exb, den.at[qq].at[rows])

        plsc.subcore_barrier()


def _edge_pass(src, dst, qs, ks, vs):
    zn = jnp.zeros((CHUNK, QW), jnp.float32)
    zd = jnp.zeros((CHUNK, LANES), jnp.float32)
    mesh = plsc.VectorSubcoreMesh(core_axis_name="c", subcore_axis_name="s")
    f = pl.kernel(
        _edge_body,
        out_type=[
            jax.ShapeDtypeStruct((4, N, QW), jnp.float32),
            jax.ShapeDtypeStruct((4, N, LANES), jnp.float32),
        ],
        mesh=mesh,
        compiler_params=pltpu.CompilerParams(needs_layout_passes=False,
                                             use_tc_tiling_on_sc=False),
        scratch_types=[
            pltpu.VMEM((EPW,), jnp.int32),
            pltpu.VMEM((EPW,), jnp.int32),
            [pltpu.VMEM((CHUNK,), jnp.int32) for _ in range(3)],
            [pltpu.VMEM((CHUNK,), jnp.int32) for _ in range(3)],
            [pltpu.VMEM((CHUNK, QW), jnp.float32) for _ in range(3)],
            [pltpu.VMEM((CHUNK, QW), jnp.float32) for _ in range(3)],
            [pltpu.VMEM((CHUNK, QW), jnp.float32) for _ in range(3)],
            pltpu.VMEM((CHUNK, QW), jnp.float32),
            pltpu.VMEM((CHUNK, LANES), jnp.float32),
            pltpu.VMEM_SHARED((N, QW), jnp.float32),
            pltpu.VMEM_SHARED((N, LANES), jnp.float32),
            pltpu.SemaphoreType.DMA,
            pltpu.SemaphoreType.DMA,
            pltpu.SemaphoreType.DMA,
        ],
    )
    return f(src, dst, qs, ks, vs, zn, zd)


def _post_body(num_ref, den_ref, h_ref, w_ref, b_ref, hc_ref,
               g_ref, bb_ref, out_ref):
    parts = []
    for qt in range(4):
        d2 = den_ref[qt][:, :HPQ]
        div = jnp.broadcast_to(d2[:, :, None], (_ROW_BLK, HPQ, D))
        div = div.reshape(_ROW_BLK, QW)
        parts.append(num_ref[qt] / (div + 1e-16))
    agg = jnp.concatenate(parts, axis=1)
    ge = agg * 0.5 * (1.0 + lax.erf(agg / math.sqrt(2.0)))
    out = jnp.dot(ge, w_ref[...], preferred_element_type=jnp.float32)
    out = out + b_ref[...] + hc_ref[0, 0] * h_ref[...]
    mu = jnp.mean(out, axis=-1, keepdims=True)
    var = jnp.mean((out - mu) ** 2, axis=-1, keepdims=True)
    out_ref[...] = (out - mu) / jnp.sqrt(var + 1e-5) * g_ref[...] + bb_ref[...]


def _post(num, den, h, w2, b2, hc, ng, nb):
    grid = (N // _ROW_BLK,)
    full = lambda shape: pl.BlockSpec(shape, lambda i: tuple(0 for _ in shape))
    return pl.pallas_call(
        _post_body,
        grid=grid,
        in_specs=[
            pl.BlockSpec((4, _ROW_BLK, QW), lambda i: (0, i, 0)),
            pl.BlockSpec((4, _ROW_BLK, LANES), lambda i: (0, i, 0)),
            pl.BlockSpec((_ROW_BLK, HID), lambda i: (i, 0)),
            full((256, 256)), full((256,)), full((1, 1)),
            full((256,)), full((256,)),
        ],
        out_specs=pl.BlockSpec((_ROW_BLK, HID), lambda i: (i, 0)),
        out_shape=jax.ShapeDtypeStruct((N, HID), jnp.float32),
    )(num, den, h, w2, b2, hc, ng, nb)


def kernel(x, edge_index, lin_W, lin_b, ln_in_g, ln_in_b, k_W, k_b, q_W, q_b,
           v_W, v_b, a_rel, m_rel, p_rel, skip, a_lin_W, a_lin_b,
           norm_g, norm_b):
    scale = p_rel / math.sqrt(D)
    qw = (q_W.T.reshape(HID, HEADS, D) * scale[None, :, None]).reshape(HID, HID)
    qb = (q_b.reshape(HEADS, D) * scale[:, None]).reshape(HID)
    kw = jnp.einsum("fhd,hde->fhe", k_W.T.reshape(HID, HEADS, D),
                    a_rel).reshape(HID, HID)
    kb = jnp.einsum("hd,hde->he", k_b.reshape(HEADS, D), a_rel).reshape(HID)
    vw = jnp.einsum("fhd,hde->fhe", v_W.T.reshape(HID, HEADS, D),
                    m_rel).reshape(HID, HID)
    vb = jnp.einsum("hd,hde->he", v_b.reshape(HEADS, D), m_rel).reshape(HID)

    h, qs, ks, vs = _pre(x, lin_W.T, lin_b, ln_in_g, ln_in_b,
                         qw, qb, kw, kb, vw, vb)

    src = edge_index[0]
    dst = edge_index[1]
    num, den = _edge_pass(src, dst, qs, ks, vs)

    a_skip = jax.nn.sigmoid(skip[0])
    w2 = a_lin_W.T * a_skip
    b2 = a_lin_b * a_skip
    hc = jnp.full((1, 1), 2.0 - a_skip, jnp.float32)

    return _post(num, den, h, w2, b2, hc, norm_g, norm_b)


# speedup vs baseline: 1.6325x; 1.1032x over previous
import math

import jax
import jax.numpy as jnp
from jax import lax
from jax.experimental import pallas as pl
from jax.experimental.pallas import tpu as pltpu
from jax.experimental.pallas import tpu_sc as plsc

N = 10000
E = 160000
HID = 256
HEADS = 8
D = HID // HEADS
QW = 64
HPQ = 2

NC = 2
NS = 16
LANES = 16

CHUNK = 80
EPW = E // NS
NCHUNK = EPW // CHUNK
NDRAIN = N // CHUNK

_ROW_BLK = 1000


def _pre_body(x_ref, linw_ref, linb_ref, lng_ref, lnb_ref, qw_ref, qb_ref,
              kw_ref, kb_ref, vw_ref, vb_ref,
              h_ref, q_ref, k_ref, v_ref):
    xb = x_ref[...]
    h = jnp.dot(xb, linw_ref[...], preferred_element_type=jnp.float32)
    h = h + linb_ref[...]
    mu = jnp.mean(h, axis=-1, keepdims=True)
    var = jnp.mean((h - mu) ** 2, axis=-1, keepdims=True)
    h = (h - mu) / jnp.sqrt(var + 1e-5) * lng_ref[...] + lnb_ref[...]
    h = jnp.maximum(h, 0.0)
    h_ref[...] = h
    q = jnp.dot(h, qw_ref[...], preferred_element_type=jnp.float32) + qb_ref[...]
    k = jnp.dot(h, kw_ref[...], preferred_element_type=jnp.float32) + kb_ref[...]
    v = jnp.dot(h, vw_ref[...], preferred_element_type=jnp.float32) + vb_ref[...]
    for qt in range(4):
        q_ref[qt] = q[:, qt * QW:(qt + 1) * QW]
        k_ref[qt] = k[:, qt * QW:(qt + 1) * QW]
        v_ref[qt] = v[:, qt * QW:(qt + 1) * QW]


def _pre(x, linw, linb, lng, lnb, qw, qb, kw, kb, vw, vb):
    grid = (N // _ROW_BLK,)
    full = lambda shape: pl.BlockSpec(shape, lambda i: tuple(0 for _ in shape))
    row = lambda w: pl.BlockSpec((_ROW_BLK, w), lambda i: (i, 0))
    quarter = pl.BlockSpec((4, _ROW_BLK, QW), lambda i: (0, i, 0))
    return pl.pallas_call(
        _pre_body,
        grid=grid,
        in_specs=[row(256), full((256, 256)), full((256,)), full((256,)),
                  full((256,)), full((256, 256)), full((256,)),
                  full((256, 256)), full((256,)), full((256, 256)),
                  full((256,))],
        out_specs=[row(256), quarter, quarter, quarter],
        out_shape=[
            jax.ShapeDtypeStruct((N, HID), jnp.float32),
            jax.ShapeDtypeStruct((4, N, QW), jnp.float32),
            jax.ShapeDtypeStruct((4, N, QW), jnp.float32),
            jax.ShapeDtypeStruct((4, N, QW), jnp.float32),
        ],
    )(x, linw, linb, lng, lnb, qw, qb, kw, kb, vw, vb)


def _edge_body(src, dst, qs, ks, vs, zn, zd, num, den,
               src_all, dst_all, sidx, didx, scidx, qrows, krows, vrows,
               msg, exb, num_s, den_s, sem_g0, sem_g1, sem_s0, sem_s1):
    c = lax.axis_index("c")
    s = lax.axis_index("s")
    lane = lax.iota(jnp.int32, LANES)
    sem_g = [sem_g0, sem_g1]
    sem_s = [sem_s0, sem_s1]

    def copy_idx(src_ref, dst_ref, j):
        for g in range(CHUNK // LANES):
            dst_ref[pl.ds(g * LANES, LANES)] = (
                src_ref[pl.ds(j * CHUNK + g * LANES, LANES)])

    def issue_gathers(qq, j, b):
        copy_idx(src_all, sidx[b], j)
        copy_idx(dst_all, didx[b], j)
        pltpu.async_copy(qs.at[qq].at[didx[b]], qrows[b], sem_g[b])
        pltpu.async_copy(ks.at[qq].at[sidx[b]], krows[b], sem_g[b])
        pltpu.async_copy(vs.at[qq].at[sidx[b]], vrows[b], sem_g[b])

    def wait_gathers(qq, b):
        pltpu.make_async_copy(qs.at[qq].at[didx[b]], qrows[b], sem_g[b]).wait()
        pltpu.make_async_copy(ks.at[qq].at[sidx[b]], krows[b], sem_g[b]).wait()
        pltpu.make_async_copy(vs.at[qq].at[sidx[b]], vrows[b], sem_g[b]).wait()

    def wait_scatters(b):
        pltpu.make_async_copy(msg[b], num_s.at[scidx[b]], sem_s[b]).wait()
        pltpu.make_async_copy(exb[b], den_s.at[scidx[b]], sem_s[b]).wait()

    def process(qq, j, b, nb, prefetch, scatter_guard):
        wait_gathers(qq, b)
        if prefetch:
            issue_gathers(qq, j + 1, nb)
        if scatter_guard is True:
            wait_scatters(b)
        elif scatter_guard is not False:
            @pl.when(scatter_guard)
            def _():
                wait_scatters(b)

        def edges(e8, _):
            for i in range(16):
                e = e8 * 16 + i
                den_acc = jnp.zeros((LANES,), jnp.float32)
                for h in range(HPQ):
                    q0 = qrows[b][e, pl.ds(h * D, LANES)]
                    q1 = qrows[b][e, pl.ds(h * D + LANES, LANES)]
                    k0 = krows[b][e, pl.ds(h * D, LANES)]
                    k1 = krows[b][e, pl.ds(h * D + LANES, LANES)]
                    pr = q0 * k0 + q1 * k1
                    a = jnp.sum(pr)
                    bc = jnp.zeros((LANES,), jnp.float32) + a
                    bc = jnp.minimum(jnp.maximum(bc, -60.0), 60.0)
                    eb = jnp.exp(bc)
                    msg[b][e, pl.ds(h * D, LANES)] = (
                        vrows[b][e, pl.ds(h * D, LANES)] * eb)
                    msg[b][e, pl.ds(h * D + LANES, LANES)] = (
                        vrows[b][e, pl.ds(h * D + LANES, LANES)] * eb)
                    den_acc = den_acc + jnp.where(lane == h, eb, 0.0)
                exb[b][e, :] = den_acc
            return 0

        lax.fori_loop(0, CHUNK // 16, edges, 0)
        copy_idx(didx[b], scidx[b], 0)
        pltpu.async_copy(msg[b], num_s.at[scidx[b]], sem_s[b], add=True)
        pltpu.async_copy(exb[b], den_s.at[scidx[b]], sem_s[b], add=True)

    for p in range(2):
        qq = c * 2 + p

        for t in range((NDRAIN + NS - 1) // NS):
            cid = s + NS * t

            @pl.when(cid < NDRAIN)
            def _():
                rows = pl.ds(cid * CHUNK, CHUNK)
                pltpu.sync_copy(zn, num_s.at[rows])
                pltpu.sync_copy(zd, den_s.at[rows])

        plsc.subcore_barrier()

        pltpu.sync_copy(src.at[pl.ds(s * EPW, EPW)], src_all)
        pltpu.sync_copy(dst.at[pl.ds(s * EPW, EPW)], dst_all)

        issue_gathers(qq, 0, 0)

        def pair(pp, _):
            j0 = pp * 2
            guard = pp >= 1
            process(qq, j0, 0, 1, True, guard)
            process(qq, j0 + 1, 1, 0, True, guard)
            return 0

        lax.fori_loop(0, NCHUNK // 2, pair, 0)
        process(qq, NCHUNK - 1, 0, 1, False, True)
        wait_scatters(0)
        wait_scatters(1)

        plsc.subcore_barrier()

        for t in range((NDRAIN + NS - 1) // NS):
            cid = s + NS * t

            @pl.when(cid < NDRAIN)
            def _():
                rows = pl.ds(cid * CHUNK, CHUNK)
                pltpu.sync_copy(num_s.at[rows], msg[0])
                pltpu.sync_copy(msg[0], num.at[qq].at[rows])
                pltpu.sync_copy(den_s.at[rows], exb[0])
                pltpu.sync_copy(exb[0], den.at[qq].at[rows])

        plsc.subcore_barrier()


def _edge_pass(src, dst, qs, ks, vs):
    zn = jnp.zeros((CHUNK, QW), jnp.float32)
    zd = jnp.zeros((CHUNK, LANES), jnp.float32)
    mesh = plsc.VectorSubcoreMesh(core_axis_name="c", subcore_axis_name="s")
    f = pl.kernel(
        _edge_body,
        out_type=[
            jax.ShapeDtypeStruct((4, N, QW), jnp.float32),
            jax.ShapeDtypeStruct((4, N, LANES), jnp.float32),
        ],
        mesh=mesh,
        compiler_params=pltpu.CompilerParams(needs_layout_passes=False,
                                             use_tc_tiling_on_sc=False),
        scratch_types=[
            pltpu.VMEM((EPW,), jnp.int32),
            pltpu.VMEM((EPW,), jnp.int32),
            [pltpu.VMEM((CHUNK,), jnp.int32) for _ in range(2)],
            [pltpu.VMEM((CHUNK,), jnp.int32) for _ in range(2)],
            [pltpu.VMEM((CHUNK,), jnp.int32) for _ in range(2)],
            [pltpu.VMEM((CHUNK, QW), jnp.float32) for _ in range(2)],
            [pltpu.VMEM((CHUNK, QW), jnp.float32) for _ in range(2)],
            [pltpu.VMEM((CHUNK, QW), jnp.float32) for _ in range(2)],
            [pltpu.VMEM((CHUNK, QW), jnp.float32) for _ in range(2)],
            [pltpu.VMEM((CHUNK, LANES), jnp.float32) for _ in range(2)],
            pltpu.VMEM_SHARED((N, QW), jnp.float32),
            pltpu.VMEM_SHARED((N, LANES), jnp.float32),
            pltpu.SemaphoreType.DMA,
            pltpu.SemaphoreType.DMA,
            pltpu.SemaphoreType.DMA,
            pltpu.SemaphoreType.DMA,
        ],
    )
    return f(src, dst, qs, ks, vs, zn, zd)


def _post_body(num_ref, den_ref, h_ref, w_ref, b_ref, hc_ref,
               g_ref, bb_ref, out_ref):
    parts = []
    for qt in range(4):
        d2 = den_ref[qt][:, :HPQ]
        div = jnp.broadcast_to(d2[:, :, None], (_ROW_BLK, HPQ, D))
        div = div.reshape(_ROW_BLK, QW)
        parts.append(num_ref[qt] / (div + 1e-16))
    agg = jnp.concatenate(parts, axis=1)
    ge = agg * 0.5 * (1.0 + lax.erf(agg / math.sqrt(2.0)))
    out = jnp.dot(ge, w_ref[...], preferred_element_type=jnp.float32)
    out = out + b_ref[...] + hc_ref[0, 0] * h_ref[...]
    mu = jnp.mean(out, axis=-1, keepdims=True)
    var = jnp.mean((out - mu) ** 2, axis=-1, keepdims=True)
    out_ref[...] = (out - mu) / jnp.sqrt(var + 1e-5) * g_ref[...] + bb_ref[...]


def _post(num, den, h, w2, b2, hc, ng, nb):
    grid = (N // _ROW_BLK,)
    full = lambda shape: pl.BlockSpec(shape, lambda i: tuple(0 for _ in shape))
    return pl.pallas_call(
        _post_body,
        grid=grid,
        in_specs=[
            pl.BlockSpec((4, _ROW_BLK, QW), lambda i: (0, i, 0)),
            pl.BlockSpec((4, _ROW_BLK, LANES), lambda i: (0, i, 0)),
            pl.BlockSpec((_ROW_BLK, HID), lambda i: (i, 0)),
            full((256, 256)), full((256,)), full((1, 1)),
            full((256,)), full((256,)),
        ],
        out_specs=pl.BlockSpec((_ROW_BLK, HID), lambda i: (i, 0)),
        out_shape=jax.ShapeDtypeStruct((N, HID), jnp.float32),
    )(num, den, h, w2, b2, hc, ng, nb)


def kernel(x, edge_index, lin_W, lin_b, ln_in_g, ln_in_b, k_W, k_b, q_W, q_b,
           v_W, v_b, a_rel, m_rel, p_rel, skip, a_lin_W, a_lin_b,
           norm_g, norm_b):
    scale = p_rel / math.sqrt(D)
    qw = (q_W.T.reshape(HID, HEADS, D) * scale[None, :, None]).reshape(HID, HID)
    qb = (q_b.reshape(HEADS, D) * scale[:, None]).reshape(HID)
    kw = jnp.einsum("fhd,hde->fhe", k_W.T.reshape(HID, HEADS, D),
                    a_rel).reshape(HID, HID)
    kb = jnp.einsum("hd,hde->he", k_b.reshape(HEADS, D), a_rel).reshape(HID)
    vw = jnp.einsum("fhd,hde->fhe", v_W.T.reshape(HID, HEADS, D),
                    m_rel).reshape(HID, HID)
    vb = jnp.einsum("hd,hde->he", v_b.reshape(HEADS, D), m_rel).reshape(HID)

    h, qs, ks, vs = _pre(x, lin_W.T, lin_b, ln_in_g, ln_in_b,
                         qw, qb, kw, kb, vw, vb)

    src = edge_index[0]
    dst = edge_index[1]
    num, den = _edge_pass(src, dst, qs, ks, vs)

    a_skip = jax.nn.sigmoid(skip[0])
    w2 = a_lin_W.T * a_skip
    b2 = a_lin_b * a_skip
    hc = jnp.full((1, 1), 2.0 - a_skip, jnp.float32)

    return _post(num, den, h, w2, b2, hc, norm_g, norm_b)


# speedup vs baseline: 1.6357x; 1.0020x over previous
import math

import jax
import jax.numpy as jnp
from jax import lax
from jax.experimental import pallas as pl
from jax.experimental.pallas import tpu as pltpu
from jax.experimental.pallas import tpu_sc as plsc

N = 10000
E = 160000
HID = 256
HEADS = 8
D = HID // HEADS
QW = 64
HPQ = 2

NC = 2
NS = 16
LANES = 16

CHUNK = 80
EPW = E // NS
NCHUNK = EPW // CHUNK
NDRAIN = N // CHUNK

_ROW_BLK = 1000


def _pre_body(x_ref, linw_ref, linb_ref, lng_ref, lnb_ref, qw_ref, qb_ref,
              kw_ref, kb_ref, vw_ref, vb_ref,
              h_ref, q_ref, k_ref, v_ref):
    xb = x_ref[...]
    h = jnp.dot(xb, linw_ref[...], preferred_element_type=jnp.float32)
    h = h + linb_ref[...]
    mu = jnp.mean(h, axis=-1, keepdims=True)
    var = jnp.mean((h - mu) ** 2, axis=-1, keepdims=True)
    h = (h - mu) / jnp.sqrt(var + 1e-5) * lng_ref[...] + lnb_ref[...]
    h = jnp.maximum(h, 0.0)
    h_ref[...] = h
    q = jnp.dot(h, qw_ref[...], preferred_element_type=jnp.float32) + qb_ref[...]
    k = jnp.dot(h, kw_ref[...], preferred_element_type=jnp.float32) + kb_ref[...]
    v = jnp.dot(h, vw_ref[...], preferred_element_type=jnp.float32) + vb_ref[...]
    for qt in range(4):
        q_ref[qt] = q[:, qt * QW:(qt + 1) * QW]
        k_ref[qt] = k[:, qt * QW:(qt + 1) * QW]
        v_ref[qt] = v[:, qt * QW:(qt + 1) * QW]


def _pre(x, linw, linb, lng, lnb, qw, qb, kw, kb, vw, vb):
    grid = (N // _ROW_BLK,)
    full = lambda shape: pl.BlockSpec(shape, lambda i: tuple(0 for _ in shape))
    row = lambda w: pl.BlockSpec((_ROW_BLK, w), lambda i: (i, 0))
    quarter = pl.BlockSpec((4, _ROW_BLK, QW), lambda i: (0, i, 0))
    return pl.pallas_call(
        _pre_body,
        grid=grid,
        in_specs=[row(256), full((256, 256)), full((256,)), full((256,)),
                  full((256,)), full((256, 256)), full((256,)),
                  full((256, 256)), full((256,)), full((256, 256)),
                  full((256,))],
        out_specs=[row(256), quarter, quarter, quarter],
        out_shape=[
            jax.ShapeDtypeStruct((N, HID), jnp.float32),
            jax.ShapeDtypeStruct((4, N, QW), jnp.float32),
            jax.ShapeDtypeStruct((4, N, QW), jnp.float32),
            jax.ShapeDtypeStruct((4, N, QW), jnp.float32),
        ],
    )(x, linw, linb, lng, lnb, qw, qb, kw, kb, vw, vb)


def _edge_body(src, dst, qs, ks, vs, zn, zd, num, den,
               src_all, dst_all, sidx, didx, scidx, qrows, krows, vrows,
               msg, exb, num_s, den_s, sem_g0, sem_g1, sem_s0, sem_s1):
    c = lax.axis_index("c")
    s = lax.axis_index("s")
    lane = lax.iota(jnp.int32, LANES)
    sem_g = [sem_g0, sem_g1]
    sem_s = [sem_s0, sem_s1]

    def copy_idx(src_ref, dst_ref, j):
        for g in range(CHUNK // LANES):
            dst_ref[pl.ds(g * LANES, LANES)] = (
                src_ref[pl.ds(j * CHUNK + g * LANES, LANES)])

    def issue_gathers(qq, j, b):
        copy_idx(src_all, sidx[b], j)
        copy_idx(dst_all, didx[b], j)
        pltpu.async_copy(qs.at[qq].at[didx[b]], qrows[b], sem_g[b])
        pltpu.async_copy(ks.at[qq].at[sidx[b]], krows[b], sem_g[b])
        pltpu.async_copy(vs.at[qq].at[sidx[b]], vrows[b], sem_g[b])

    def wait_gathers(qq, b):
        pltpu.make_async_copy(qs.at[qq].at[didx[b]], qrows[b], sem_g[b]).wait()
        pltpu.make_async_copy(ks.at[qq].at[sidx[b]], krows[b], sem_g[b]).wait()
        pltpu.make_async_copy(vs.at[qq].at[sidx[b]], vrows[b], sem_g[b]).wait()

    def wait_scatters(b):
        pltpu.make_async_copy(msg[b], num_s.at[scidx[b]], sem_s[b]).wait()
        pltpu.make_async_copy(exb[b], den_s.at[scidx[b]], sem_s[b]).wait()

    def process(qq, j, b, nb, prefetch, scatter_guard):
        wait_gathers(qq, b)
        if prefetch:
            issue_gathers(qq, j + 1, nb)
        if scatter_guard is True:
            wait_scatters(b)
        elif scatter_guard is not False:
            @pl.when(scatter_guard)
            def _():
                wait_scatters(b)

        def edges(e8, _):
            for i in range(8):
                e = e8 * 8 + i
                den_acc = jnp.zeros((LANES,), jnp.float32)
                for h in range(HPQ):
                    q0 = qrows[b][e, pl.ds(h * D, LANES)]
                    q1 = qrows[b][e, pl.ds(h * D + LANES, LANES)]
                    k0 = krows[b][e, pl.ds(h * D, LANES)]
                    k1 = krows[b][e, pl.ds(h * D + LANES, LANES)]
                    pr = q0 * k0 + q1 * k1
                    a = jnp.sum(pr)
                    bc = jnp.zeros((LANES,), jnp.float32) + a
                    bc = jnp.minimum(jnp.maximum(bc, -60.0), 60.0)
                    eb = jnp.exp(bc)
                    msg[b][e, pl.ds(h * D, LANES)] = (
                        vrows[b][e, pl.ds(h * D, LANES)] * eb)
                    msg[b][e, pl.ds(h * D + LANES, LANES)] = (
                        vrows[b][e, pl.ds(h * D + LANES, LANES)] * eb)
                    den_acc = den_acc + jnp.where(lane == h, eb, 0.0)
                exb[b][e, :] = den_acc
            return 0

        lax.fori_loop(0, CHUNK // 8, edges, 0)
        copy_idx(didx[b], scidx[b], 0)
        pltpu.async_copy(msg[b], num_s.at[scidx[b]], sem_s[b], add=True)
        pltpu.async_copy(exb[b], den_s.at[scidx[b]], sem_s[b], add=True)

    for p in range(2):
        qq = c * 2 + p

        for t in range((NDRAIN + NS - 1) // NS):
            cid = s + NS * t

            @pl.when(cid < NDRAIN)
            def _():
                rows = pl.ds(cid * CHUNK, CHUNK)
                pltpu.sync_copy(zn, num_s.at[rows])
                pltpu.sync_copy(zd, den_s.at[rows])

        plsc.subcore_barrier()

        pltpu.sync_copy(src.at[pl.ds(s * EPW, EPW)], src_all)
        pltpu.sync_copy(dst.at[pl.ds(s * EPW, EPW)], dst_all)

        issue_gathers(qq, 0, 0)

        def pair(pp, _):
            j0 = pp * 2
            guard = pp >= 1
            process(qq, j0, 0, 1, True, guard)
            process(qq, j0 + 1, 1, 0, True, guard)
            return 0

        lax.fori_loop(0, NCHUNK // 2, pair, 0)
        process(qq, NCHUNK - 1, 0, 1, False, True)
        wait_scatters(0)
        wait_scatters(1)

        plsc.subcore_barrier()

        for t in range((NDRAIN + NS - 1) // NS):
            cid = s + NS * t

            @pl.when(cid < NDRAIN)
            def _():
                rows = pl.ds(cid * CHUNK, CHUNK)
                pltpu.sync_copy(num_s.at[rows], msg[0])
                pltpu.sync_copy(msg[0], num.at[qq].at[rows])
                pltpu.sync_copy(den_s.at[rows], exb[0])
                pltpu.sync_copy(exb[0], den.at[qq].at[rows])

        plsc.subcore_barrier()


def _edge_pass(src, dst, qs, ks, vs):
    zn = jnp.zeros((CHUNK, QW), jnp.float32)
    zd = jnp.zeros((CHUNK, LANES), jnp.float32)
    mesh = plsc.VectorSubcoreMesh(core_axis_name="c", subcore_axis_name="s")
    f = pl.kernel(
        _edge_body,
        out_type=[
            jax.ShapeDtypeStruct((4, N, QW), jnp.float32),
            jax.ShapeDtypeStruct((4, N, LANES), jnp.float32),
        ],
        mesh=mesh,
        compiler_params=pltpu.CompilerParams(needs_layout_passes=False,
                                             use_tc_tiling_on_sc=False),
        scratch_types=[
            pltpu.VMEM((EPW,), jnp.int32),
            pltpu.VMEM((EPW,), jnp.int32),
            [pltpu.VMEM((CHUNK,), jnp.int32) for _ in range(2)],
            [pltpu.VMEM((CHUNK,), jnp.int32) for _ in range(2)],
            [pltpu.VMEM((CHUNK,), jnp.int32) for _ in range(2)],
            [pltpu.VMEM((CHUNK, QW), jnp.float32) for _ in range(2)],
            [pltpu.VMEM((CHUNK, QW), jnp.float32) for _ in range(2)],
            [pltpu.VMEM((CHUNK, QW), jnp.float32) for _ in range(2)],
            [pltpu.VMEM((CHUNK, QW), jnp.float32) for _ in range(2)],
            [pltpu.VMEM((CHUNK, LANES), jnp.float32) for _ in range(2)],
            pltpu.VMEM_SHARED((N, QW), jnp.float32),
            pltpu.VMEM_SHARED((N, LANES), jnp.float32),
            pltpu.SemaphoreType.DMA,
            pltpu.SemaphoreType.DMA,
            pltpu.SemaphoreType.DMA,
            pltpu.SemaphoreType.DMA,
        ],
    )
    return f(src, dst, qs, ks, vs, zn, zd)


def _post_body(num_ref, den_ref, h_ref, w_ref, b_ref, hc_ref,
               g_ref, bb_ref, out_ref):
    parts = []
    for qt in range(4):
        d2 = den_ref[qt][:, :HPQ]
        div = jnp.broadcast_to(d2[:, :, None], (_ROW_BLK, HPQ, D))
        div = div.reshape(_ROW_BLK, QW)
        parts.append(num_ref[qt] / (div + 1e-16))
    agg = jnp.concatenate(parts, axis=1)
    ge = agg * 0.5 * (1.0 + lax.erf(agg / math.sqrt(2.0)))
    out = jnp.dot(ge, w_ref[...], preferred_element_type=jnp.float32)
    out = out + b_ref[...] + hc_ref[0, 0] * h_ref[...]
    mu = jnp.mean(out, axis=-1, keepdims=True)
    var = jnp.mean((out - mu) ** 2, axis=-1, keepdims=True)
    out_ref[...] = (out - mu) / jnp.sqrt(var + 1e-5) * g_ref[...] + bb_ref[...]


def _post(num, den, h, w2, b2, hc, ng, nb):
    grid = (N // _ROW_BLK,)
    full = lambda shape: pl.BlockSpec(shape, lambda i: tuple(0 for _ in shape))
    return pl.pallas_call(
        _post_body,
        grid=grid,
        in_specs=[
            pl.BlockSpec((4, _ROW_BLK, QW), lambda i: (0, i, 0)),
            pl.BlockSpec((4, _ROW_BLK, LANES), lambda i: (0, i, 0)),
            pl.BlockSpec((_ROW_BLK, HID), lambda i: (i, 0)),
            full((256, 256)), full((256,)), full((1, 1)),
            full((256,)), full((256,)),
        ],
        out_specs=pl.BlockSpec((_ROW_BLK, HID), lambda i: (i, 0)),
        out_shape=jax.ShapeDtypeStruct((N, HID), jnp.float32),
    )(num, den, h, w2, b2, hc, ng, nb)


def kernel(x, edge_index, lin_W, lin_b, ln_in_g, ln_in_b, k_W, k_b, q_W, q_b,
           v_W, v_b, a_rel, m_rel, p_rel, skip, a_lin_W, a_lin_b,
           norm_g, norm_b):
    scale = p_rel / math.sqrt(D)
    qw = (q_W.T.reshape(HID, HEADS, D) * scale[None, :, None]).reshape(HID, HID)
    qb = (q_b.reshape(HEADS, D) * scale[:, None]).reshape(HID)
    kw = jnp.einsum("fhd,hde->fhe", k_W.T.reshape(HID, HEADS, D),
                    a_rel).reshape(HID, HID)
    kb = jnp.einsum("hd,hde->he", k_b.reshape(HEADS, D), a_rel).reshape(HID)
    vw = jnp.einsum("fhd,hde->fhe", v_W.T.reshape(HID, HEADS, D),
                    m_rel).reshape(HID, HID)
    vb = jnp.einsum("hd,hde->he", v_b.reshape(HEADS, D), m_rel).reshape(HID)

    h, qs, ks, vs = _pre(x, lin_W.T, lin_b, ln_in_g, ln_in_b,
                         qw, qb, kw, kb, vw, vb)

    src = edge_index[0]
    dst = edge_index[1]
    num, den = _edge_pass(src, dst, qs, ks, vs)

    a_skip = jax.nn.sigmoid(skip[0])
    w2 = a_lin_W.T * a_skip
    b2 = a_lin_b * a_skip
    hc = jnp.full((1, 1), 2.0 - a_skip, jnp.float32)

    return _post(num, den, h, w2, b2, hc, norm_g, norm_b)
